# baseline (speedup 1.0000x reference)
import functools

import jax
import jax.numpy as jnp
from jax import lax
from jax.experimental import pallas as pl
from jax.experimental.pallas import tpu as pltpu
from jax.experimental.pallas import tpu_sc as plsc

NC = 2
NS = 16
CH = 128
NB = 4
NIB = 8
LOOK = 2
BLK = 1000


def _cdiv(a, b):
    return -(-a // b)


def _sc_mesh():
    return plsc.VectorSubcoreMesh(core_axis_name="c", subcore_axis_name="s")


_SC_PARAMS = pltpu.CompilerParams(use_tc_tiling_on_sc=False)


@functools.lru_cache(maxsize=None)
def _deg_kernel(n_pad, nch):
    rpt = n_pad // NS
    assert nch % NB == 0

    @functools.partial(
        pl.kernel,
        out_type=jax.ShapeDtypeStruct((NC, n_pad, 64), jnp.float32),
        mesh=_sc_mesh(),
        compiler_params=_SC_PARAMS,
        scratch_types=[
            pltpu.VMEM_SHARED((n_pad, 64), jnp.float32),
            pltpu.VMEM((nch, CH), jnp.int32),
            pltpu.VMEM((CH, 64), jnp.float32),
        ] + [pltpu.SemaphoreType.DMA] * NB,
    )
    def deg_k(coli, zeros, ones, out, acc, colv, onev, *ssems):
        c = lax.axis_index("c")
        s = lax.axis_index("s")
        pltpu.sync_copy(coli.at[c, s], colv)
        pltpu.sync_copy(ones, onev)
        pltpu.sync_copy(zeros.at[pl.ds(s * rpt, rpt)], acc.at[pl.ds(s * rpt, rpt)])
        plsc.subcore_barrier()

        for b in range(NB):
            pltpu.async_copy(onev, acc.at[colv.at[b]], ssems[b], add=True)

        def outer(jo, carry):
            for b in range(NB):
                j = jo * NB + b + NB
                pltpu.make_async_copy(onev, acc.at[colv.at[j]], ssems[b]).wait()
                pltpu.async_copy(onev, acc.at[colv.at[j]], ssems[b], add=True)
            return carry

        lax.fori_loop(0, nch // NB - 1, outer, 0)
        for b in range(NB):
            pltpu.make_async_copy(onev, acc.at[colv.at[b]], ssems[b]).wait()
        plsc.subcore_barrier()
        pltpu.sync_copy(acc.at[pl.ds(s * rpt, rpt)], out.at[c, pl.ds(s * rpt, rpt)])

    return deg_k


@functools.lru_cache(maxsize=None)
def _hop_kernel(n, n_pad, nch):
    rpt = n_pad // NS
    assert nch % NIB == 0

    @functools.partial(
        pl.kernel,
        out_type=jax.ShapeDtypeStruct((NC, n_pad, 64), jnp.float32),
        mesh=_sc_mesh(),
        compiler_params=_SC_PARAMS,
        scratch_types=[
            pltpu.VMEM_SHARED((n_pad, 64), jnp.float32),
            pltpu.VMEM_SHARED((n_pad, 64), jnp.float32),
            pltpu.VMEM((NIB, 3, CH), jnp.int32),
            pltpu.VMEM((NB, CH, 64), jnp.float32),
        ] + [pltpu.SemaphoreType.DMA] * (NIB + 2 * NB),
    )
    def hop_k(z2, z2f, idxs, zeros, out, zsh, acc, ring, buf, *sems):
        isems = sems[:NIB]
        gsems = sems[NIB:NIB + NB]
        ssems = sems[NIB + NB:]
        c = lax.axis_index("c")
        s = lax.axis_index("s")
        pltpu.sync_copy(z2.at[c, pl.ds(s * rpt, rpt)], zsh.at[pl.ds(s * rpt, rpt)])
        pltpu.sync_copy(zeros.at[pl.ds(s * rpt, rpt)], acc.at[pl.ds(s * rpt, rpt)])
        plsc.subcore_barrier()

        def idx_issue(q, slot):
            pltpu.async_copy(idxs.at[c, s, q], ring.at[slot], isems[slot])

        def idx_wait(q, slot):
            pltpu.make_async_copy(idxs.at[c, s, q], ring.at[slot], isems[slot]).wait()

        def g_issue(b, slot, hbm):
            if hbm:
                pltpu.async_copy(z2f.at[ring.at[slot, 1]], buf.at[b], gsems[b])
            else:
                pltpu.async_copy(zsh.at[ring.at[slot, 0]], buf.at[b], gsems[b])

        def g_wait(b, slot, hbm):
            if hbm:
                pltpu.make_async_copy(
                    z2f.at[ring.at[slot, 1]], buf.at[b], gsems[b]).wait()
            else:
                pltpu.make_async_copy(
                    zsh.at[ring.at[slot, 0]], buf.at[b], gsems[b]).wait()

        def s_issue(b, slot):
            pltpu.async_copy(buf.at[b], acc.at[ring.at[slot, 2]], ssems[b], add=True)

        def s_wait(b, slot):
            pltpu.make_async_copy(buf.at[b], acc.at[ring.at[slot, 2]], ssems[b]).wait()

        for q in range(NIB - LOOK):
            idx_issue(q, q)
        for jj in range(LOOK):
            idx_wait(jj, jj)
            g_issue(jj % NB, jj % NIB, hbm=(jj % 2 == 1))

        def outer(jo, carry):
            for u in range(NIB):
                j = jo * NIB + u
                b = u % NB
                b2 = (u + LOOK) % NB
                sl2 = (u + LOOK) % NIB
                g_wait(b, u, hbm=(u % 2 == 1))
                s_issue(b, u)
                j2 = j + LOOK

                @pl.when(j2 >= NB)
                def _():
                    s_wait(b2, (u + LOOK - NB) % NIB)

                @pl.when(j + NIB - LOOK < nch)
                def _():
                    idx_issue(j + NIB - LOOK, (u + NIB - LOOK) % NIB)

                @pl.when(j2 < nch)
                def _():
                    idx_wait(j2, sl2)
                    g_issue(b2, sl2, hbm=(sl2 % 2 == 1))
            return carry

        lax.fori_loop(0, nch // NIB, outer, 0)
        for t in range(NB - LOOK):
            j = nch - NB + LOOK + t
            s_wait(j % NB, j % NIB)
        plsc.subcore_barrier()
        pltpu.sync_copy(acc.at[pl.ds(s * rpt, rpt)], out.at[c, pl.ds(s * rpt, rpt)])

    return hop_k


def _mlp_call(x, W1, b1, W2, b2, dparts, temp, n_pad):
    n, d = x.shape
    dh = d // 2
    grid = (n // BLK,)
    full = pl.BlockSpec((BLK, d), lambda i: (i, 0))
    wspec = pl.BlockSpec((d, d), lambda i: (0, 0))
    bspec = pl.BlockSpec((1, d), lambda i: (0, 0))
    sspec = pl.BlockSpec((NC, BLK, dh), lambda i: (0, i, 0))
    tspec = pl.BlockSpec(memory_space=pltpu.SMEM)

    def body(x_ref, w1_ref, b1_ref, w2_ref, b2_ref, d_ref, t_ref,
             z0_ref, hacc0_ref, dinv2_ref, dinv_ref):
        h = jnp.dot(x_ref[...], w1_ref[...], preferred_element_type=jnp.float32)
        h = jnp.maximum(h + b1_ref[...], 0.0)
        h = jnp.dot(h, w2_ref[...], preferred_element_type=jnp.float32) + b2_ref[...]
        deg = d_ref[0, :, 0:1] + d_ref[1, :, 0:1] + 1.0
        y0 = jnp.sqrt(deg) * h
        z0 = (1.0 / deg) * y0
        z0_ref[0] = z0[:, :dh]
        z0_ref[1] = z0[:, dh:]
        hacc0_ref[...] = t_ref[0] * y0
        dinv2_ref[...] = jnp.broadcast_to(1.0 / deg, h.shape)
        dinv_ref[...] = jnp.broadcast_to(lax.rsqrt(deg), h.shape)

    out_t = [
        jax.ShapeDtypeStruct((NC, n_pad, dh), jnp.float32),
        jax.ShapeDtypeStruct((n, d), jnp.float32),
        jax.ShapeDtypeStruct((n, d), jnp.float32),
        jax.ShapeDtypeStruct((n, d), jnp.float32),
    ]
    return pl.pallas_call(
        body,
        grid=grid,
        in_specs=[full, wspec, bspec, wspec, bspec, sspec, tspec],
        out_specs=[sspec, full, full, full],
        out_shape=out_t,
    )(x, W1, b1, W2, b2, dparts, temp)


def _hop_tc(rparts, z, hacc, scale, temp, k, last, n_pad):
    n, d = hacc.shape
    dh = d // 2
    grid = (n // BLK,)
    full = pl.BlockSpec((BLK, d), lambda i: (i, 0))
    sspec = pl.BlockSpec((NC, BLK, dh), lambda i: (0, i, 0))
    tspec = pl.BlockSpec(memory_space=pltpu.SMEM)

    if last:
        def body(r_ref, z_ref, hacc_ref, s_ref, t_ref, out_ref):
            y = jnp.concatenate(
                [r_ref[0] + z_ref[0], r_ref[1] + z_ref[1]], axis=-1)
            out_ref[...] = s_ref[...] * (hacc_ref[...] + t_ref[k] * y)

        return pl.pallas_call(
            body,
            grid=grid,
            in_specs=[sspec, sspec, full, full, tspec],
            out_specs=full,
            out_shape=jax.ShapeDtypeStruct((n, d), jnp.float32),
        )(rparts, z, hacc, scale, temp)

    def body(r_ref, z_ref, hacc_ref, s_ref, t_ref, z_out, hacc_out):
        y = jnp.concatenate(
            [r_ref[0] + z_ref[0], r_ref[1] + z_ref[1]], axis=-1)
        hacc_out[...] = hacc_ref[...] + t_ref[k] * y
        zs = s_ref[...] * y
        z_out[0] = zs[:, :dh]
        z_out[1] = zs[:, dh:]

    return pl.pallas_call(
        body,
        grid=grid,
        in_specs=[sspec, sspec, full, full, tspec],
        out_specs=[sspec, full],
        out_shape=[
            jax.ShapeDtypeStruct((NC, n_pad, dh), jnp.float32),
            jax.ShapeDtypeStruct((n, d), jnp.float32),
        ],
    )(rparts, z, hacc, scale, temp)


def kernel(x, edge_index, W1, b1, W2, b2, temp):
    n, d = x.shape
    dh = d // 2
    e = edge_index.shape[1]
    k_hops = temp.shape[0] - 1
    n_pad = _cdiv(n + 1, NS * 8) * (NS * 8)

    per_tile_d = e // (NC * NS)
    nch_d = _cdiv(_cdiv(per_tile_d, CH), NB) * NB
    pad_d = nch_d * CH - per_tile_d
    col_d = edge_index[1].reshape(NC, NS, per_tile_d)
    col_d = jnp.pad(col_d, ((0, 0), (0, 0), (0, pad_d)), constant_values=n)
    coli_d = col_d.reshape(NC, NS, nch_d, CH)

    per_tile = e // NS
    nch = _cdiv(_cdiv(per_tile, CH), NIB) * NIB
    pad = nch * CH - per_tile
    row_h = jnp.pad(edge_index[0].reshape(NS, per_tile), ((0, 0), (0, pad)))
    col_h = jnp.pad(edge_index[1].reshape(NS, per_tile), ((0, 0), (0, pad)),
                    constant_values=n)
    row_h = row_h.reshape(NS, nch, CH)
    col_h = col_h.reshape(NS, nch, CH)
    offs = (jnp.arange(NC, dtype=jnp.int32) * n_pad)[:, None, None, None]
    idxs = jnp.stack([
        jnp.broadcast_to(row_h[None], (NC, NS, nch, CH)),
        jnp.broadcast_to(row_h[None] + offs, (NC, NS, nch, CH)),
        jnp.broadcast_to(col_h[None], (NC, NS, nch, CH)),
    ], axis=3)

    zeros64 = jnp.zeros((n_pad, dh), jnp.float32)
    ones = jnp.ones((CH, 64), jnp.float32)

    dparts = _deg_kernel(n_pad, nch_d)(coli_d, zeros64, ones)
    z, hacc, dinv2b, dinvb = _mlp_call(
        x, W1, b1.reshape(1, d), W2, b2.reshape(1, d), dparts, temp, n_pad)

    hop = _hop_kernel(n, n_pad, nch)
    for k in range(1, k_hops + 1):
        rparts = hop(z, z.reshape(NC * n_pad, dh), idxs, zeros64)
        if k < k_hops:
            z, hacc = _hop_tc(rparts, z, hacc, dinv2b, temp, k, last=False,
                              n_pad=n_pad)
        else:
            hidden = _hop_tc(rparts, z, hacc, dinvb, temp, k, last=True,
                             n_pad=n_pad)
    return hidden


# speedup vs baseline: 1.5581x; 1.5581x over previous
import functools

import jax
import jax.numpy as jnp
from jax import lax
from jax.experimental import pallas as pl
from jax.experimental.pallas import tpu as pltpu
from jax.experimental.pallas import tpu_sc as plsc

NC = 2
NS = 16
CH = 128
NB = 4
NIB = 8
LOOK = 2
BLK = 1000


def _cdiv(a, b):
    return -(-a // b)


def _sc_mesh():
    return plsc.VectorSubcoreMesh(core_axis_name="c", subcore_axis_name="s")


_SC_PARAMS = pltpu.CompilerParams(use_tc_tiling_on_sc=False)


@functools.lru_cache(maxsize=None)
def _deg_kernel(n_pad, nch):
    rpt = n_pad // NS
    assert nch % NB == 0

    @functools.partial(
        pl.kernel,
        out_type=jax.ShapeDtypeStruct((NC, n_pad, 64), jnp.float32),
        mesh=_sc_mesh(),
        compiler_params=_SC_PARAMS,
        scratch_types=[
            pltpu.VMEM_SHARED((n_pad, 64), jnp.float32),
            pltpu.VMEM((nch, CH), jnp.int32),
            pltpu.VMEM((CH, 64), jnp.float32),
        ] + [pltpu.SemaphoreType.DMA] * NB,
    )
    def deg_k(coli, zeros, ones, out, acc, colv, onev, *ssems):
        c = lax.axis_index("c")
        s = lax.axis_index("s")
        pltpu.sync_copy(coli.at[c, s], colv)
        pltpu.sync_copy(ones, onev)
        pltpu.sync_copy(zeros.at[pl.ds(s * rpt, rpt)], acc.at[pl.ds(s * rpt, rpt)])
        plsc.subcore_barrier()

        for b in range(NB):
            pltpu.async_copy(onev, acc.at[colv.at[b]], ssems[b], add=True)

        def outer(jo, carry):
            for b in range(NB):
                j = jo * NB + b + NB
                pltpu.make_async_copy(onev, acc.at[colv.at[j]], ssems[b]).wait()
                pltpu.async_copy(onev, acc.at[colv.at[j]], ssems[b], add=True)
            return carry

        lax.fori_loop(0, nch // NB - 1, outer, 0)
        for b in range(NB):
            pltpu.make_async_copy(onev, acc.at[colv.at[b]], ssems[b]).wait()
        plsc.subcore_barrier()
        pltpu.sync_copy(acc.at[pl.ds(s * rpt, rpt)], out.at[c, pl.ds(s * rpt, rpt)])

    return deg_k


@functools.lru_cache(maxsize=None)
def _hop_kernel(n, n_pad, nch):
    rpt = n_pad // NS
    assert nch % NIB == 0

    @functools.partial(
        pl.kernel,
        out_type=jax.ShapeDtypeStruct((NC, n_pad, 64), jnp.float32),
        mesh=_sc_mesh(),
        compiler_params=_SC_PARAMS,
        scratch_types=[
            pltpu.VMEM_SHARED((n_pad, 64), jnp.float32),
            pltpu.VMEM_SHARED((n_pad, 64), jnp.float32),
            pltpu.VMEM((NIB, 2, CH), jnp.int32),
            pltpu.VMEM((NB, CH, 64), jnp.float32),
        ] + [pltpu.SemaphoreType.DMA] * (NIB + 2 * NB),
    )
    def hop_k(z2, idxs, zeros, out, zsh, acc, ring, buf, *sems):
        isems = sems[:NIB]
        gsems = sems[NIB:NIB + NB]
        ssems = sems[NIB + NB:]
        c = lax.axis_index("c")
        s = lax.axis_index("s")
        pltpu.sync_copy(z2.at[c, pl.ds(s * rpt, rpt)], zsh.at[pl.ds(s * rpt, rpt)])
        pltpu.sync_copy(zeros.at[pl.ds(s * rpt, rpt)], acc.at[pl.ds(s * rpt, rpt)])
        plsc.subcore_barrier()

        def idx_issue(q, slot):
            pltpu.async_copy(idxs.at[s, q], ring.at[slot], isems[slot])

        def idx_wait(q, slot):
            pltpu.make_async_copy(idxs.at[s, q], ring.at[slot], isems[slot]).wait()

        def g_issue(b, slot):
            pltpu.async_copy(zsh.at[ring.at[slot, 0]], buf.at[b], gsems[b])

        def g_wait(b, slot):
            pltpu.make_async_copy(zsh.at[ring.at[slot, 0]], buf.at[b], gsems[b]).wait()

        def s_issue(b, slot):
            pltpu.async_copy(buf.at[b], acc.at[ring.at[slot, 1]], ssems[b], add=True)

        def s_wait(b, slot):
            pltpu.make_async_copy(buf.at[b], acc.at[ring.at[slot, 1]], ssems[b]).wait()

        for q in range(NIB - LOOK):
            idx_issue(q, q)
        for jj in range(LOOK):
            idx_wait(jj, jj)
            g_issue(jj % NB, jj % NIB)

        def outer(jo, carry):
            for u in range(NIB):
                j = jo * NIB + u
                b = u % NB
                b2 = (u + LOOK) % NB
                sl2 = (u + LOOK) % NIB
                g_wait(b, u)
                s_issue(b, u)
                j2 = j + LOOK

                @pl.when(j2 >= NB)
                def _():
                    s_wait(b2, (u + LOOK - NB) % NIB)

                @pl.when(j + NIB - LOOK < nch)
                def _():
                    idx_issue(j + NIB - LOOK, (u + NIB - LOOK) % NIB)

                @pl.when(j2 < nch)
                def _():
                    idx_wait(j2, sl2)
                    g_issue(b2, sl2)
            return carry

        lax.fori_loop(0, nch // NIB, outer, 0)
        for t in range(NB - LOOK):
            j = nch - NB + LOOK + t
            s_wait(j % NB, j % NIB)
        plsc.subcore_barrier()
        pltpu.sync_copy(acc.at[pl.ds(s * rpt, rpt)], out.at[c, pl.ds(s * rpt, rpt)])

    return hop_k


def _mlp_call(x, W1, b1, W2, b2, dparts, temp, n_pad):
    n, d = x.shape
    dh = d // 2
    grid = (n // BLK,)
    full = pl.BlockSpec((BLK, d), lambda i: (i, 0))
    wspec = pl.BlockSpec((d, d), lambda i: (0, 0))
    bspec = pl.BlockSpec((1, d), lambda i: (0, 0))
    sspec = pl.BlockSpec((NC, BLK, dh), lambda i: (0, i, 0))
    tspec = pl.BlockSpec(memory_space=pltpu.SMEM)

    def body(x_ref, w1_ref, b1_ref, w2_ref, b2_ref, d_ref, t_ref,
             z0_ref, hacc0_ref, dinv2_ref, dinv_ref):
        h = jnp.dot(x_ref[...], w1_ref[...], preferred_element_type=jnp.float32)
        h = jnp.maximum(h + b1_ref[...], 0.0)
        h = jnp.dot(h, w2_ref[...], preferred_element_type=jnp.float32) + b2_ref[...]
        deg = d_ref[0, :, 0:1] + d_ref[1, :, 0:1] + 1.0
        y0 = jnp.sqrt(deg) * h
        z0 = (1.0 / deg) * y0
        z0_ref[0] = z0[:, :dh]
        z0_ref[1] = z0[:, dh:]
        hacc0_ref[...] = t_ref[0] * y0
        dinv2_ref[...] = jnp.broadcast_to(1.0 / deg, h.shape)
        dinv_ref[...] = jnp.broadcast_to(lax.rsqrt(deg), h.shape)

    out_t = [
        jax.ShapeDtypeStruct((NC, n_pad, dh), jnp.float32),
        jax.ShapeDtypeStruct((n, d), jnp.float32),
        jax.ShapeDtypeStruct((n, d), jnp.float32),
        jax.ShapeDtypeStruct((n, d), jnp.float32),
    ]
    return pl.pallas_call(
        body,
        grid=grid,
        in_specs=[full, wspec, bspec, wspec, bspec, sspec, tspec],
        out_specs=[sspec, full, full, full],
        out_shape=out_t,
    )(x, W1, b1, W2, b2, dparts, temp)


def _hop_tc(rparts, z, hacc, scale, temp, k, last, n_pad):
    n, d = hacc.shape
    dh = d // 2
    grid = (n // BLK,)
    full = pl.BlockSpec((BLK, d), lambda i: (i, 0))
    sspec = pl.BlockSpec((NC, BLK, dh), lambda i: (0, i, 0))
    tspec = pl.BlockSpec(memory_space=pltpu.SMEM)

    if last:
        def body(r_ref, z_ref, hacc_ref, s_ref, t_ref, out_ref):
            y = jnp.concatenate(
                [r_ref[0] + z_ref[0], r_ref[1] + z_ref[1]], axis=-1)
            out_ref[...] = s_ref[...] * (hacc_ref[...] + t_ref[k] * y)

        return pl.pallas_call(
            body,
            grid=grid,
            in_specs=[sspec, sspec, full, full, tspec],
            out_specs=full,
            out_shape=jax.ShapeDtypeStruct((n, d), jnp.float32),
        )(rparts, z, hacc, scale, temp)

    def body(r_ref, z_ref, hacc_ref, s_ref, t_ref, z_out, hacc_out):
        y = jnp.concatenate(
            [r_ref[0] + z_ref[0], r_ref[1] + z_ref[1]], axis=-1)
        hacc_out[...] = hacc_ref[...] + t_ref[k] * y
        zs = s_ref[...] * y
        z_out[0] = zs[:, :dh]
        z_out[1] = zs[:, dh:]

    return pl.pallas_call(
        body,
        grid=grid,
        in_specs=[sspec, sspec, full, full, tspec],
        out_specs=[sspec, full],
        out_shape=[
            jax.ShapeDtypeStruct((NC, n_pad, dh), jnp.float32),
            jax.ShapeDtypeStruct((n, d), jnp.float32),
        ],
    )(rparts, z, hacc, scale, temp)


def kernel(x, edge_index, W1, b1, W2, b2, temp):
    n, d = x.shape
    dh = d // 2
    e = edge_index.shape[1]
    k_hops = temp.shape[0] - 1
    n_pad = _cdiv(n + 1, NS * 8) * (NS * 8)

    per_tile_d = e // (NC * NS)
    nch_d = _cdiv(_cdiv(per_tile_d, CH), NB) * NB
    pad_d = nch_d * CH - per_tile_d
    col_d = edge_index[1].reshape(NC, NS, per_tile_d)
    col_d = jnp.pad(col_d, ((0, 0), (0, 0), (0, pad_d)), constant_values=n)
    coli_d = col_d.reshape(NC, NS, nch_d, CH)

    per_tile = e // NS
    nch = _cdiv(_cdiv(per_tile, CH), NIB) * NIB
    pad = nch * CH - per_tile
    row_h = jnp.pad(edge_index[0].reshape(NS, per_tile), ((0, 0), (0, pad)))
    col_h = jnp.pad(edge_index[1].reshape(NS, per_tile), ((0, 0), (0, pad)),
                    constant_values=n)
    idxs = jnp.stack([row_h.reshape(NS, nch, CH), col_h.reshape(NS, nch, CH)],
                     axis=2)

    zeros64 = jnp.zeros((n_pad, dh), jnp.float32)
    ones = jnp.ones((CH, 64), jnp.float32)

    dparts = _deg_kernel(n_pad, nch_d)(coli_d, zeros64, ones)
    z, hacc, dinv2b, dinvb = _mlp_call(
        x, W1, b1.reshape(1, d), W2, b2.reshape(1, d), dparts, temp, n_pad)

    hop = _hop_kernel(n, n_pad, nch)
    for k in range(1, k_hops + 1):
        rparts = hop(z, idxs, zeros64)
        if k < k_hops:
            z, hacc = _hop_tc(rparts, z, hacc, dinv2b, temp, k, last=False,
                              n_pad=n_pad)
        else:
            hidden = _hop_tc(rparts, z, hacc, dinvb, temp, k, last=True,
                             n_pad=n_pad)
    return hidden


# speedup vs baseline: 1.5819x; 1.0153x over previous
import functools

import jax
import jax.numpy as jnp
from jax import lax
from jax.experimental import pallas as pl
from jax.experimental.pallas import tpu as pltpu
from jax.experimental.pallas import tpu_sc as plsc

NC = 2
NS = 16
CH = 128
NB = 4
NIB = 8
LOOK = 2
BLK = 1000


def _cdiv(a, b):
    return -(-a // b)


def _sc_mesh():
    return plsc.VectorSubcoreMesh(core_axis_name="c", subcore_axis_name="s")


_SC_PARAMS = pltpu.CompilerParams(use_tc_tiling_on_sc=False)


@functools.lru_cache(maxsize=None)
def _deg_kernel(n_pad, nch):
    rpt = n_pad // NS
    assert nch % NB == 0

    @functools.partial(
        pl.kernel,
        out_type=jax.ShapeDtypeStruct((NC, n_pad, 64), jnp.float32),
        mesh=_sc_mesh(),
        compiler_params=_SC_PARAMS,
        scratch_types=[
            pltpu.VMEM_SHARED((n_pad, 64), jnp.float32),
            pltpu.VMEM((nch, CH), jnp.int32),
            pltpu.VMEM((CH, 64), jnp.float32),
        ] + [pltpu.SemaphoreType.DMA] * NB,
    )
    def deg_k(coli, zeros, ones, out, acc, colv, onev, *ssems):
        c = lax.axis_index("c")
        s = lax.axis_index("s")
        pltpu.sync_copy(coli.at[c, s], colv)
        pltpu.sync_copy(ones, onev)
        pltpu.sync_copy(zeros.at[pl.ds(s * rpt, rpt)], acc.at[pl.ds(s * rpt, rpt)])
        plsc.subcore_barrier()

        for b in range(NB):
            pltpu.async_copy(onev, acc.at[colv.at[b]], ssems[b], add=True)

        def outer(jo, carry):
            for b in range(NB):
                j = jo * NB + b + NB
                pltpu.make_async_copy(onev, acc.at[colv.at[j]], ssems[b]).wait()
                pltpu.async_copy(onev, acc.at[colv.at[j]], ssems[b], add=True)
            return carry

        lax.fori_loop(0, nch // NB - 1, outer, 0)
        for b in range(NB):
            pltpu.make_async_copy(onev, acc.at[colv.at[b]], ssems[b]).wait()
        plsc.subcore_barrier()
        pltpu.sync_copy(acc.at[pl.ds(s * rpt, rpt)], out.at[c, pl.ds(s * rpt, rpt)])

    return deg_k


@functools.lru_cache(maxsize=None)
def _hop_kernel(n, n_pad, nch):
    rpt = n_pad // NS
    assert nch % NIB == 0

    @functools.partial(
        pl.kernel,
        out_type=jax.ShapeDtypeStruct((NC, n_pad, 64), jnp.float32),
        mesh=_sc_mesh(),
        compiler_params=_SC_PARAMS,
        scratch_types=[
            pltpu.VMEM_SHARED((n_pad, 64), jnp.float32),
            pltpu.VMEM_SHARED((n_pad, 64), jnp.float32),
            pltpu.VMEM((NIB, 2, CH), jnp.int32),
            pltpu.VMEM((NB, CH, 64), jnp.float32),
        ] + [pltpu.SemaphoreType.DMA] * (NIB + 2 * NB),
    )
    def hop_k(z2, idxs, zeros, out, zsh, acc, ring, buf, *sems):
        isems = sems[:NIB]
        gsems = sems[NIB:NIB + NB]
        ssems = sems[NIB + NB:]
        c = lax.axis_index("c")
        s = lax.axis_index("s")
        pltpu.sync_copy(z2.at[c, pl.ds(s * rpt, rpt)], zsh.at[pl.ds(s * rpt, rpt)])
        pltpu.sync_copy(zeros.at[pl.ds(s * rpt, rpt)], acc.at[pl.ds(s * rpt, rpt)])
        plsc.subcore_barrier()

        def idx_issue(q, slot):
            pltpu.async_copy(idxs.at[s, q], ring.at[slot], isems[slot])

        def idx_wait(q, slot):
            pltpu.make_async_copy(idxs.at[s, q], ring.at[slot], isems[slot]).wait()

        def g_issue(b, slot):
            pltpu.async_copy(zsh.at[ring.at[slot, 0]], buf.at[b], gsems[b])

        def g_wait(b, slot):
            pltpu.make_async_copy(zsh.at[ring.at[slot, 0]], buf.at[b], gsems[b]).wait()

        def s_issue(b, slot):
            pltpu.async_copy(buf.at[b], acc.at[ring.at[slot, 1]], ssems[b], add=True)

        def s_wait(b, slot):
            pltpu.make_async_copy(buf.at[b], acc.at[ring.at[slot, 1]], ssems[b]).wait()

        for q in range(NIB - LOOK):
            idx_issue(q, q)
        for jj in range(LOOK):
            idx_wait(jj, jj)
            g_issue(jj % NB, jj % NIB)

        def outer(jo, carry):
            for u in range(NIB):
                j = jo * NIB + u
                b = u % NB
                b2 = (u + LOOK) % NB
                sl2 = (u + LOOK) % NIB
                g_wait(b, u)
                s_issue(b, u)
                j2 = j + LOOK

                @pl.when(j2 >= NB)
                def _():
                    s_wait(b2, (u + LOOK - NB) % NIB)

                @pl.when(j + NIB - LOOK < nch)
                def _():
                    idx_issue(j + NIB - LOOK, (u + NIB - LOOK) % NIB)

                @pl.when(j2 < nch)
                def _():
                    idx_wait(j2, sl2)
                    g_issue(b2, sl2)
            return carry

        lax.fori_loop(0, nch // NIB, outer, 0)
        for t in range(NB - LOOK):
            j = nch - NB + LOOK + t
            s_wait(j % NB, j % NIB)
        plsc.subcore_barrier()
        pltpu.sync_copy(acc.at[pl.ds(s * rpt, rpt)], out.at[c, pl.ds(s * rpt, rpt)])

    return hop_k


def _mlp_call(x, W1, b1, W2, b2, dparts, temp, n_pad):
    n, d = x.shape
    dh = d // 2
    grid = (n // BLK,)
    full = pl.BlockSpec((BLK, d), lambda i: (i, 0))
    wspec = pl.BlockSpec((d, d), lambda i: (0, 0))
    bspec = pl.BlockSpec((1, d), lambda i: (0, 0))
    sspec = pl.BlockSpec((NC, BLK, dh), lambda i: (0, i, 0))
    tspec = pl.BlockSpec(memory_space=pltpu.SMEM)

    def body(x_ref, w1_ref, b1_ref, w2_ref, b2_ref, d_ref, t_ref,
             z0_ref, hacc0_ref, dinv2_ref, dinv_ref):
        h = jnp.dot(x_ref[...], w1_ref[...], preferred_element_type=jnp.float32)
        h = jnp.maximum(h + b1_ref[...], 0.0)
        h = jnp.dot(h, w2_ref[...], preferred_element_type=jnp.float32) + b2_ref[...]
        deg = d_ref[0, :, 0:1] + d_ref[1, :, 0:1] + 1.0
        y0 = jnp.sqrt(deg) * h
        z0 = (1.0 / deg) * y0
        z0_ref[0] = z0[:, :dh]
        z0_ref[1] = z0[:, dh:]
        hacc0_ref[...] = t_ref[0] * y0
        dinv2_ref[...] = jnp.broadcast_to(1.0 / deg, (deg.shape[0], 8))
        dinv_ref[...] = jnp.broadcast_to(lax.rsqrt(deg), (deg.shape[0], 8))

    out_t = [
        jax.ShapeDtypeStruct((NC, n_pad, dh), jnp.float32),
        jax.ShapeDtypeStruct((n, d), jnp.float32),
        jax.ShapeDtypeStruct((n, 8), jnp.float32),
        jax.ShapeDtypeStruct((n, 8), jnp.float32),
    ]
    nspec = pl.BlockSpec((BLK, 8), lambda i: (i, 0))
    return pl.pallas_call(
        body,
        grid=grid,
        in_specs=[full, wspec, bspec, wspec, bspec, sspec, tspec],
        out_specs=[sspec, full, nspec, nspec],
        out_shape=out_t,
    )(x, W1, b1, W2, b2, dparts, temp)


def _hop_z(rparts, z, scale, n_pad, n, d):
    dh = d // 2
    grid = (n // BLK,)
    sspec = pl.BlockSpec((NC, BLK, dh), lambda i: (0, i, 0))
    nspec = pl.BlockSpec((BLK, 8), lambda i: (i, 0))

    def body(r_ref, z_ref, s_ref, z_out):
        sc = s_ref[:, 0:1]
        z_out[0] = sc * (r_ref[0] + z_ref[0])
        z_out[1] = sc * (r_ref[1] + z_ref[1])

    return pl.pallas_call(
        body,
        grid=grid,
        in_specs=[sspec, sspec, nspec],
        out_specs=sspec,
        out_shape=jax.ShapeDtypeStruct((NC, n_pad, dh), jnp.float32),
    )(rparts, z, scale)


def _hop_hacc(rparts, z, hacc, temp, k):
    n, d = hacc.shape
    dh = d // 2
    grid = (n // BLK,)
    full = pl.BlockSpec((BLK, d), lambda i: (i, 0))
    sspec = pl.BlockSpec((NC, BLK, dh), lambda i: (0, i, 0))
    tspec = pl.BlockSpec(memory_space=pltpu.SMEM)

    def body(r_ref, z_ref, hacc_ref, t_ref, hacc_out):
        y = jnp.concatenate(
            [r_ref[0] + z_ref[0], r_ref[1] + z_ref[1]], axis=-1)
        hacc_out[...] = hacc_ref[...] + t_ref[k] * y

    return pl.pallas_call(
        body,
        grid=grid,
        in_specs=[sspec, sspec, full, tspec],
        out_specs=full,
        out_shape=jax.ShapeDtypeStruct((n, d), jnp.float32),
    )(rparts, z, hacc, temp)


def _hop_final(rparts, z, hacc, scale, temp, k):
    n, d = hacc.shape
    dh = d // 2
    grid = (n // BLK,)
    full = pl.BlockSpec((BLK, d), lambda i: (i, 0))
    sspec = pl.BlockSpec((NC, BLK, dh), lambda i: (0, i, 0))
    nspec = pl.BlockSpec((BLK, 8), lambda i: (i, 0))
    tspec = pl.BlockSpec(memory_space=pltpu.SMEM)

    def body(r_ref, z_ref, hacc_ref, s_ref, t_ref, out_ref):
        y = jnp.concatenate(
            [r_ref[0] + z_ref[0], r_ref[1] + z_ref[1]], axis=-1)
        out_ref[...] = s_ref[:, 0:1] * (hacc_ref[...] + t_ref[k] * y)

    return pl.pallas_call(
        body,
        grid=grid,
        in_specs=[sspec, sspec, full, nspec, tspec],
        out_specs=full,
        out_shape=jax.ShapeDtypeStruct((n, d), jnp.float32),
    )(rparts, z, hacc, scale, temp)


def kernel(x, edge_index, W1, b1, W2, b2, temp):
    n, d = x.shape
    dh = d // 2
    e = edge_index.shape[1]
    k_hops = temp.shape[0] - 1
    n_pad = _cdiv(n + 1, NS * 8) * (NS * 8)

    per_tile_d = e // (NC * NS)
    nch_d = _cdiv(_cdiv(per_tile_d, CH), NB) * NB
    pad_d = nch_d * CH - per_tile_d
    col_d = edge_index[1].reshape(NC, NS, per_tile_d)
    col_d = jnp.pad(col_d, ((0, 0), (0, 0), (0, pad_d)), constant_values=n)
    coli_d = col_d.reshape(NC, NS, nch_d, CH)

    per_tile = e // NS
    nch = _cdiv(_cdiv(per_tile, CH), NIB) * NIB
    pad = nch * CH - per_tile
    row_h = jnp.pad(edge_index[0].reshape(NS, per_tile), ((0, 0), (0, pad)))
    col_h = jnp.pad(edge_index[1].reshape(NS, per_tile), ((0, 0), (0, pad)),
                    constant_values=n)
    idxs = jnp.stack([row_h.reshape(NS, nch, CH), col_h.reshape(NS, nch, CH)],
                     axis=2)

    zeros64 = jnp.zeros((n_pad, dh), jnp.float32)
    ones = jnp.ones((CH, 64), jnp.float32)

    dparts = _deg_kernel(n_pad, nch_d)(coli_d, zeros64, ones)
    z, hacc, dinv2b, dinvb = _mlp_call(
        x, W1, b1.reshape(1, d), W2, b2.reshape(1, d), dparts, temp, n_pad)

    hop = _hop_kernel(n, n_pad, nch)
    for k in range(1, k_hops + 1):
        rparts = hop(z, idxs, zeros64)
        if k < k_hops:
            z_new = _hop_z(rparts, z, dinv2b, n_pad, n, d)
            hacc = _hop_hacc(rparts, z, hacc, temp, k)
            z = z_new
        else:
            hidden = _hop_final(rparts, z, hacc, dinvb, temp, k)
    return hidden


# speedup vs baseline: 1.6047x; 1.0144x over previous
import functools

import jax
import jax.numpy as jnp
from jax import lax
from jax.experimental import pallas as pl
from jax.experimental.pallas import tpu as pltpu
from jax.experimental.pallas import tpu_sc as plsc

NC = 2
NS = 16
CH = 128
NB = 4
NIB = 8
LOOK = 2
BLK = 2000


def _cdiv(a, b):
    return -(-a // b)


def _sc_mesh():
    return plsc.VectorSubcoreMesh(core_axis_name="c", subcore_axis_name="s")


_SC_PARAMS = pltpu.CompilerParams(use_tc_tiling_on_sc=False)


@functools.lru_cache(maxsize=None)
def _deg_kernel(n_pad, nch):
    rpt = n_pad // NS
    assert nch % NB == 0

    @functools.partial(
        pl.kernel,
        out_type=jax.ShapeDtypeStruct((NC, n_pad, 64), jnp.float32),
        mesh=_sc_mesh(),
        compiler_params=_SC_PARAMS,
        scratch_types=[
            pltpu.VMEM_SHARED((n_pad, 64), jnp.float32),
            pltpu.VMEM((nch, CH), jnp.int32),
            pltpu.VMEM((CH, 64), jnp.float32),
        ] + [pltpu.SemaphoreType.DMA] * NB,
    )
    def deg_k(coli, zeros, ones, out, acc, colv, onev, *ssems):
        c = lax.axis_index("c")
        s = lax.axis_index("s")
        pltpu.sync_copy(coli.at[c, s], colv)
        pltpu.sync_copy(ones, onev)
        pltpu.sync_copy(zeros.at[pl.ds(s * rpt, rpt)], acc.at[pl.ds(s * rpt, rpt)])
        plsc.subcore_barrier()

        for b in range(NB):
            pltpu.async_copy(onev, acc.at[colv.at[b]], ssems[b], add=True)

        def outer(jo, carry):
            for b in range(NB):
                j = jo * NB + b + NB
                pltpu.make_async_copy(onev, acc.at[colv.at[j]], ssems[b]).wait()
                pltpu.async_copy(onev, acc.at[colv.at[j]], ssems[b], add=True)
            return carry

        lax.fori_loop(0, nch // NB - 1, outer, 0)
        for b in range(NB):
            pltpu.make_async_copy(onev, acc.at[colv.at[b]], ssems[b]).wait()
        plsc.subcore_barrier()
        pltpu.sync_copy(acc.at[pl.ds(s * rpt, rpt)], out.at[c, pl.ds(s * rpt, rpt)])

    return deg_k


@functools.lru_cache(maxsize=None)
def _hop_kernel(n, n_pad, nch):
    rpt = n_pad // NS
    assert nch % NIB == 0

    @functools.partial(
        pl.kernel,
        out_type=jax.ShapeDtypeStruct((NC, n_pad, 64), jnp.float32),
        mesh=_sc_mesh(),
        compiler_params=_SC_PARAMS,
        scratch_types=[
            pltpu.VMEM_SHARED((n_pad, 64), jnp.float32),
            pltpu.VMEM_SHARED((n_pad, 64), jnp.float32),
            pltpu.VMEM((NIB, 2, CH), jnp.int32),
            pltpu.VMEM((NB, CH, 64), jnp.float32),
        ] + [pltpu.SemaphoreType.DMA] * (NIB + 2 * NB),
    )
    def hop_k(z2, idxs, zeros, out, zsh, acc, ring, buf, *sems):
        isems = sems[:NIB]
        gsems = sems[NIB:NIB + NB]
        ssems = sems[NIB + NB:]
        c = lax.axis_index("c")
        s = lax.axis_index("s")

        def idx_issue(q, slot):
            pltpu.async_copy(idxs.at[s, q], ring.at[slot], isems[slot])

        def idx_wait(q, slot):
            pltpu.make_async_copy(idxs.at[s, q], ring.at[slot], isems[slot]).wait()

        def g_issue(b, slot):
            pltpu.async_copy(zsh.at[ring.at[slot, 0]], buf.at[b], gsems[b])

        def g_wait(b, slot):
            pltpu.make_async_copy(zsh.at[ring.at[slot, 0]], buf.at[b], gsems[b]).wait()

        def s_issue(b, slot):
            pltpu.async_copy(buf.at[b], acc.at[ring.at[slot, 1]], ssems[b], add=True)

        def s_wait(b, slot):
            pltpu.make_async_copy(buf.at[b], acc.at[ring.at[slot, 1]], ssems[b]).wait()

        for q in range(NIB - LOOK):
            idx_issue(q, q)
        pltpu.sync_copy(z2.at[c, pl.ds(s * rpt, rpt)], zsh.at[pl.ds(s * rpt, rpt)])
        pltpu.sync_copy(zeros.at[pl.ds(s * rpt, rpt)], acc.at[pl.ds(s * rpt, rpt)])
        plsc.subcore_barrier()
        for jj in range(LOOK):
            idx_wait(jj, jj)
            g_issue(jj % NB, jj % NIB)

        def outer(jo, carry):
            for u in range(NIB):
                j = jo * NIB + u
                b = u % NB
                b2 = (u + LOOK) % NB
                sl2 = (u + LOOK) % NIB
                g_wait(b, u)
                s_issue(b, u)
                j2 = j + LOOK

                @pl.when(j2 >= NB)
                def _():
                    s_wait(b2, (u + LOOK - NB) % NIB)

                @pl.when(j + NIB - LOOK < nch)
                def _():
                    idx_issue(j + NIB - LOOK, (u + NIB - LOOK) % NIB)

                @pl.when(j2 < nch)
                def _():
                    idx_wait(j2, sl2)
                    g_issue(b2, sl2)
            return carry

        lax.fori_loop(0, nch // NIB, outer, 0)
        for t in range(NB - LOOK):
            j = nch - NB + LOOK + t
            s_wait(j % NB, j % NIB)
        plsc.subcore_barrier()
        pltpu.sync_copy(acc.at[pl.ds(s * rpt, rpt)], out.at[c, pl.ds(s * rpt, rpt)])

    return hop_k


def _mlp_call(x, W1, b1, W2, b2, dparts, temp, n_pad):
    n, d = x.shape
    dh = d // 2
    grid = (n // BLK,)
    full = pl.BlockSpec((BLK, d), lambda i: (i, 0))
    wspec = pl.BlockSpec((d, d), lambda i: (0, 0))
    bspec = pl.BlockSpec((1, d), lambda i: (0, 0))
    sspec = pl.BlockSpec((NC, BLK, dh), lambda i: (0, i, 0))
    tspec = pl.BlockSpec(memory_space=pltpu.SMEM)

    def body(x_ref, w1_ref, b1_ref, w2_ref, b2_ref, d_ref, t_ref,
             z0_ref, hacc0_ref, dinv2_ref, dinv_ref):
        h = jnp.dot(x_ref[...], w1_ref[...], preferred_element_type=jnp.float32)
        h = jnp.maximum(h + b1_ref[...], 0.0)
        h = jnp.dot(h, w2_ref[...], preferred_element_type=jnp.float32) + b2_ref[...]
        deg = d_ref[0, :, 0:1] + d_ref[1, :, 0:1] + 1.0
        y0 = jnp.sqrt(deg) * h
        z0 = (1.0 / deg) * y0
        z0_ref[0] = z0[:, :dh]
        z0_ref[1] = z0[:, dh:]
        hacc0_ref[...] = t_ref[0] * y0
        dinv2_ref[...] = jnp.broadcast_to(1.0 / deg, (deg.shape[0], 8))
        dinv_ref[...] = jnp.broadcast_to(lax.rsqrt(deg), (deg.shape[0], 8))

    out_t = [
        jax.ShapeDtypeStruct((NC, n_pad, dh), jnp.float32),
        jax.ShapeDtypeStruct((n, d), jnp.float32),
        jax.ShapeDtypeStruct((n, 8), jnp.float32),
        jax.ShapeDtypeStruct((n, 8), jnp.float32),
    ]
    nspec = pl.BlockSpec((BLK, 8), lambda i: (i, 0))
    return pl.pallas_call(
        body,
        grid=grid,
        in_specs=[full, wspec, bspec, wspec, bspec, sspec, tspec],
        out_specs=[sspec, full, nspec, nspec],
        out_shape=out_t,
    )(x, W1, b1, W2, b2, dparts, temp)


def _hop_z(rparts, z, scale, n_pad, n, d):
    dh = d // 2
    grid = (n // BLK,)
    sspec = pl.BlockSpec((NC, BLK, dh), lambda i: (0, i, 0))
    nspec = pl.BlockSpec((BLK, 8), lambda i: (i, 0))

    def body(r_ref, z_ref, s_ref, z_out):
        sc = s_ref[:, 0:1]
        z_out[0] = sc * (r_ref[0] + z_ref[0])
        z_out[1] = sc * (r_ref[1] + z_ref[1])

    return pl.pallas_call(
        body,
        grid=grid,
        in_specs=[sspec, sspec, nspec],
        out_specs=sspec,
        out_shape=jax.ShapeDtypeStruct((NC, n_pad, dh), jnp.float32),
    )(rparts, z, scale)


def _hop_hacc(rparts, z, hacc, temp, k):
    n, d = hacc.shape
    dh = d // 2
    grid = (n // BLK,)
    full = pl.BlockSpec((BLK, d), lambda i: (i, 0))
    sspec = pl.BlockSpec((NC, BLK, dh), lambda i: (0, i, 0))
    tspec = pl.BlockSpec(memory_space=pltpu.SMEM)

    def body(r_ref, z_ref, hacc_ref, t_ref, hacc_out):
        y = jnp.concatenate(
            [r_ref[0] + z_ref[0], r_ref[1] + z_ref[1]], axis=-1)
        hacc_out[...] = hacc_ref[...] + t_ref[k] * y

    return pl.pallas_call(
        body,
        grid=grid,
        in_specs=[sspec, sspec, full, tspec],
        out_specs=full,
        out_shape=jax.ShapeDtypeStruct((n, d), jnp.float32),
    )(rparts, z, hacc, temp)


def _hop_final(rparts, z, hacc, scale, temp, k):
    n, d = hacc.shape
    dh = d // 2
    grid = (n // BLK,)
    full = pl.BlockSpec((BLK, d), lambda i: (i, 0))
    sspec = pl.BlockSpec((NC, BLK, dh), lambda i: (0, i, 0))
    nspec = pl.BlockSpec((BLK, 8), lambda i: (i, 0))
    tspec = pl.BlockSpec(memory_space=pltpu.SMEM)

    def body(r_ref, z_ref, hacc_ref, s_ref, t_ref, out_ref):
        y = jnp.concatenate(
            [r_ref[0] + z_ref[0], r_ref[1] + z_ref[1]], axis=-1)
        out_ref[...] = s_ref[:, 0:1] * (hacc_ref[...] + t_ref[k] * y)

    return pl.pallas_call(
        body,
        grid=grid,
        in_specs=[sspec, sspec, full, nspec, tspec],
        out_specs=full,
        out_shape=jax.ShapeDtypeStruct((n, d), jnp.float32),
    )(rparts, z, hacc, scale, temp)


def kernel(x, edge_index, W1, b1, W2, b2, temp):
    n, d = x.shape
    dh = d // 2
    e = edge_index.shape[1]
    k_hops = temp.shape[0] - 1
    n_pad = _cdiv(n + 1, NS * 8) * (NS * 8)

    per_tile_d = e // (NC * NS)
    nch_d = _cdiv(_cdiv(per_tile_d, CH), NB) * NB
    pad_d = nch_d * CH - per_tile_d
    col_d = edge_index[1].reshape(NC, NS, per_tile_d)
    col_d = jnp.pad(col_d, ((0, 0), (0, 0), (0, pad_d)), constant_values=n)
    coli_d = col_d.reshape(NC, NS, nch_d, CH)

    per_tile = e // NS
    nch = _cdiv(_cdiv(per_tile, CH), NIB) * NIB
    pad = nch * CH - per_tile
    row_h = jnp.pad(edge_index[0].reshape(NS, per_tile), ((0, 0), (0, pad)))
    col_h = jnp.pad(edge_index[1].reshape(NS, per_tile), ((0, 0), (0, pad)),
                    constant_values=n)
    idxs = jnp.stack([row_h.reshape(NS, nch, CH), col_h.reshape(NS, nch, CH)],
                     axis=2)

    zeros64 = jnp.zeros((n_pad, dh), jnp.float32)
    ones = jnp.ones((CH, 64), jnp.float32)

    dparts = _deg_kernel(n_pad, nch_d)(coli_d, zeros64, ones)
    z, hacc, dinv2b, dinvb = _mlp_call(
        x, W1, b1.reshape(1, d), W2, b2.reshape(1, d), dparts, temp, n_pad)

    hop = _hop_kernel(n, n_pad, nch)
    for k in range(1, k_hops + 1):
        rparts = hop(z, idxs, zeros64)
        if k < k_hops:
            z_new = _hop_z(rparts, z, dinv2b, n_pad, n, d)
            hacc = _hop_hacc(rparts, z, hacc, temp, k)
            z = z_new
        else:
            hidden = _hop_final(rparts, z, hacc, dinvb, temp, k)
    return hidden
